# baseline (speedup 1.0000x reference)
import jax
import jax.numpy as jnp
from jax.experimental import pallas as pl

_UPG = 2


def _est_body(xflat_ref, alrbd_ref, x_ref, st_ref, w_ref, win_ref, b_ref,
              wout_ref, sr_ref, temp_ref, ns_ref, out_ref):
    g = pl.program_id(0)
    nu = pl.num_programs(0) * _UPG
    temp = temp_ref[0, 0]

    logits = jnp.dot(xflat_ref[...], alrbd_ref[...],
                     preferred_element_type=jnp.float32) / temp
    lr = jax.nn.softmax(logits, axis=1)

    D = x_ref.shape[1] // _UPG
    N = st_ref.shape[1] // _UPG
    for j in range(_UPG):
        u = g * _UPG + j
        onehot = (jax.lax.broadcasted_iota(jnp.int32, (1, nu), 1) == u
                  ).astype(jnp.float32)
        lr_u = jnp.sum(lr * onehot, axis=1)[:, None]
        sr_u = jnp.sum(sr_ref[...] * onehot)
        x_u = x_ref[:, j * D:(j + 1) * D]
        st_u = st_ref[:, j * N:(j + 1) * N]
        bf = jnp.bfloat16
        feed = jnp.dot(x_u.astype(bf), win_ref[j].astype(bf),
                       preferred_element_type=jnp.float32)
        echo = jnp.dot((st_u * sr_u).astype(bf), w_ref[j].astype(bf),
                       preferred_element_type=jnp.float32)
        act = jnp.tanh(feed + echo + b_ref[j, 0, :][None, :])
        ns = (1.0 - lr_u) * st_u + lr_u * act
        ns_ref[:, j * N:(j + 1) * N] = ns
        out = jnp.dot(ns.astype(bf), wout_ref[j].astype(bf),
                      preferred_element_type=jnp.float32)
        out_ref[:, j * out.shape[1]:(j + 1) * out.shape[1]] = out


def kernel(X, state, W, Win, bias, Wout, sr, adaptive_lr, temperature,
           w_h, w_o, w_d, win_h, win_o, win_d):
    B, U, D = X.shape
    N = state.shape[2]
    O = Wout.shape[2]
    G = U // _UPG
    X_flat = X.reshape(B, U * D)
    st_flat = state.reshape(B, U * N)
    unit_of_row = jnp.repeat(jnp.arange(U), D)
    alr_bd = (adaptive_lr.reshape(U * D)[:, None] *
              (unit_of_row[:, None] == jnp.arange(U)[None, :]))
    sr2 = sr.reshape(1, U)
    temp2 = temperature.reshape(1, 1)
    P = _UPG
    ns, out = pl.pallas_call(
        _est_body,
        grid=(G,),
        in_specs=[
            pl.BlockSpec((B, U * D), lambda g: (0, 0)),
            pl.BlockSpec((U * D, U), lambda g: (0, 0)),
            pl.BlockSpec((B, P * D), lambda g: (0, g)),
            pl.BlockSpec((B, P * N), lambda g: (0, g)),
            pl.BlockSpec((P, N, N), lambda g: (g, 0, 0)),
            pl.BlockSpec((P, D, N), lambda g: (g, 0, 0)),
            pl.BlockSpec((P, 1, N), lambda g: (g, 0, 0)),
            pl.BlockSpec((P, N, O), lambda g: (g, 0, 0)),
            pl.BlockSpec((1, U), lambda g: (0, 0)),
            pl.BlockSpec((1, 1), lambda g: (0, 0)),
        ],
        out_specs=[
            pl.BlockSpec((B, P * N), lambda g: (0, g)),
            pl.BlockSpec((B, P * O), lambda g: (0, g)),
        ],
        out_shape=[
            jax.ShapeDtypeStruct((B, U * N), jnp.float32),
            jax.ShapeDtypeStruct((B, U * O), jnp.float32),
        ],
    )(X_flat, alr_bd, X_flat, st_flat, W, Win, bias, Wout, sr2, temp2)
    return ns.reshape(B, U, N), out.reshape(B, U, O)


# speedup vs baseline: 1.0026x; 1.0026x over previous
import jax
import jax.numpy as jnp
from jax.experimental import pallas as pl

_UPG = 2


def _est_body(xflat_ref, alrbd_ref, x_ref, st_ref, w_ref, win_ref, b_ref,
              wout_ref, sr_ref, temp_ref, ns_ref, out_ref):
    g = pl.program_id(0)
    nu = pl.num_programs(0) * _UPG
    temp = temp_ref[0, 0]

    logits = jnp.dot(xflat_ref[...], alrbd_ref[...],
                     preferred_element_type=jnp.float32) / temp
    lr = jax.nn.softmax(logits, axis=1)

    D = x_ref.shape[1] // _UPG
    N = st_ref.shape[1] // _UPG
    for j in range(_UPG):
        u = g * _UPG + j
        onehot = (jax.lax.broadcasted_iota(jnp.int32, (1, nu), 1) == u
                  ).astype(jnp.float32)
        lr_u = jnp.sum(lr * onehot, axis=1)[:, None]
        sr_u = jnp.sum(sr_ref[...] * onehot)
        x_u = x_ref[:, j * D:(j + 1) * D]
        st_u = st_ref[:, j * N:(j + 1) * N]
        feed = jnp.dot(x_u, win_ref[j], preferred_element_type=jnp.float32)
        echo = jnp.dot(st_u * sr_u, w_ref[j],
                       preferred_element_type=jnp.float32)
        act = jnp.tanh(feed + echo + b_ref[j, 0, :][None, :])
        ns = (1.0 - lr_u) * st_u + lr_u * act
        ns_ref[:, j * N:(j + 1) * N] = ns
        out = jnp.dot(ns, wout_ref[j], preferred_element_type=jnp.float32)
        out_ref[:, j * out.shape[1]:(j + 1) * out.shape[1]] = out


def kernel(X, state, W, Win, bias, Wout, sr, adaptive_lr, temperature,
           w_h, w_o, w_d, win_h, win_o, win_d):
    B, U, D = X.shape
    N = state.shape[2]
    O = Wout.shape[2]
    G = U // _UPG
    X_flat = X.reshape(B, U * D)
    st_flat = state.reshape(B, U * N)
    unit_of_row = jnp.repeat(jnp.arange(U), D)
    alr_bd = (adaptive_lr.reshape(U * D)[:, None] *
              (unit_of_row[:, None] == jnp.arange(U)[None, :]))
    sr2 = sr.reshape(1, U)
    temp2 = temperature.reshape(1, 1)
    P = _UPG
    ns, out = pl.pallas_call(
        _est_body,
        grid=(G,),
        in_specs=[
            pl.BlockSpec((B, U * D), lambda g: (0, 0)),
            pl.BlockSpec((U * D, U), lambda g: (0, 0)),
            pl.BlockSpec((B, P * D), lambda g: (0, g)),
            pl.BlockSpec((B, P * N), lambda g: (0, g)),
            pl.BlockSpec((P, N, N), lambda g: (g, 0, 0)),
            pl.BlockSpec((P, D, N), lambda g: (g, 0, 0)),
            pl.BlockSpec((P, 1, N), lambda g: (g, 0, 0)),
            pl.BlockSpec((P, N, O), lambda g: (g, 0, 0)),
            pl.BlockSpec((1, U), lambda g: (0, 0)),
            pl.BlockSpec((1, 1), lambda g: (0, 0)),
        ],
        out_specs=[
            pl.BlockSpec((B, P * N), lambda g: (0, g)),
            pl.BlockSpec((B, P * O), lambda g: (0, g)),
        ],
        out_shape=[
            jax.ShapeDtypeStruct((B, U * N), jnp.float32),
            jax.ShapeDtypeStruct((B, U * O), jnp.float32),
        ],
    )(X_flat, alr_bd, X_flat, st_flat, W, Win, bias, Wout, sr2, temp2)
    return ns.reshape(B, U, N), out.reshape(B, U, O)
